# Initial kernel scaffold; baseline (speedup 1.0000x reference)
import jax
import jax.numpy as jnp
from jax.experimental import pallas as pl


def kernel(x1, x2):
    raise NotImplementedError("write your pallas kernel here")


# speedup vs baseline: 462.2431x; 462.2431x over previous
import functools

import jax
import jax.numpy as jnp
from jax.experimental import pallas as pl
from jax.experimental.pallas import tpu as pltpu

M1 = 4096
M2 = 4096
D = 256
TOL = 1e-8

BM = 256
BN = 1024


def _tile_body(x1_ref, x2_ref, k_ref, keep_ref):
    j = pl.program_id(0)
    i = pl.program_id(1)
    a = x1_ref[...]
    b = x2_ref[...]
    n1 = jnp.sum(a * a, axis=1)
    n2 = jnp.sum(b * b, axis=1)
    prod = jax.lax.dot_general(
        a, b, (((1,), (1,)), ((), ())),
        preferred_element_type=jnp.float32,
        precision=jax.lax.Precision.HIGHEST,
    )
    sq = n1[:, None] + n2[None, :] - 2.0 * prod
    sq = jnp.maximum(sq, 0.0)
    k = jnp.exp(-0.5 * sq)
    k_ref[...] = k

    rows = i * BM + jax.lax.broadcasted_iota(jnp.int32, (BM, BN), 0)
    cols = j * BN + jax.lax.broadcasted_iota(jnp.int32, (BM, BN), 1)
    dup = ((1.0 - k) < TOL) & (rows <= cols)
    keep_tile = jnp.logical_not(jnp.any(dup, axis=0))[None, :].astype(jnp.int32)

    @pl.when(i == 0)
    def _init():
        keep_ref[...] = keep_tile

    @pl.when(i > 0)
    def _acc():
        keep_ref[...] = keep_ref[...] & keep_tile


@jax.jit
def kernel(x1, x2):
    grid = (M2 // BN, M1 // BM)
    k_mat, keep_i32 = pl.pallas_call(
        _tile_body,
        grid=grid,
        in_specs=[
            pl.BlockSpec((BM, D), lambda j, i: (i, 0)),
            pl.BlockSpec((BN, D), lambda j, i: (j, 0)),
        ],
        out_specs=[
            pl.BlockSpec((BM, BN), lambda j, i: (i, j)),
            pl.BlockSpec((1, BN), lambda j, i: (0, j)),
        ],
        out_shape=[
            jax.ShapeDtypeStruct((M1, M2), jnp.float32),
            jax.ShapeDtypeStruct((1, M2), jnp.int32),
        ],
        compiler_params=pltpu.CompilerParams(
            dimension_semantics=("parallel", "arbitrary"),
        ),
    )(x1, x2)
    keep_mask = keep_i32[0].astype(bool)
    return k_mat, keep_mask


# speedup vs baseline: 682.6378x; 1.4768x over previous
import functools

import jax
import jax.numpy as jnp
from jax.experimental import pallas as pl
from jax.experimental.pallas import tpu as pltpu

M1 = 4096
M2 = 4096
D = 256
TOL = 1e-8

BM = 256
BN = 1024


def _tile_body(x1_ref, x2_ref, k_ref, keep_ref):
    j = pl.program_id(0)
    i = pl.program_id(1)
    a = x1_ref[...]
    b = x2_ref[...]
    n1 = jnp.sum(a * a, axis=1)
    n2 = jnp.sum(b * b, axis=1)
    prod = jax.lax.dot_general(
        a, b, (((1,), (1,)), ((), ())),
        preferred_element_type=jnp.float32,
        precision=jax.lax.Precision.DEFAULT,
    )
    sq = n1[:, None] + n2[None, :] - 2.0 * prod
    sq = jnp.maximum(sq, 0.0)
    k = jnp.exp(-0.5 * sq)
    k_ref[...] = k

    rows = i * BM + jax.lax.broadcasted_iota(jnp.int32, (BM, BN), 0)
    cols = j * BN + jax.lax.broadcasted_iota(jnp.int32, (BM, BN), 1)
    dup = ((1.0 - k) < TOL) & (rows <= cols)
    keep_tile = jnp.logical_not(jnp.any(dup, axis=0))[None, :].astype(jnp.int32)

    @pl.when(i == 0)
    def _init():
        keep_ref[...] = keep_tile

    @pl.when(i > 0)
    def _acc():
        keep_ref[...] = keep_ref[...] & keep_tile


@jax.jit
def kernel(x1, x2):
    grid = (M2 // BN, M1 // BM)
    k_mat, keep_i32 = pl.pallas_call(
        _tile_body,
        grid=grid,
        in_specs=[
            pl.BlockSpec((BM, D), lambda j, i: (i, 0)),
            pl.BlockSpec((BN, D), lambda j, i: (j, 0)),
        ],
        out_specs=[
            pl.BlockSpec((BM, BN), lambda j, i: (i, j)),
            pl.BlockSpec((1, BN), lambda j, i: (0, j)),
        ],
        out_shape=[
            jax.ShapeDtypeStruct((M1, M2), jnp.float32),
            jax.ShapeDtypeStruct((1, M2), jnp.int32),
        ],
        compiler_params=pltpu.CompilerParams(
            dimension_semantics=("parallel", "arbitrary"),
        ),
    )(x1, x2)
    keep_mask = keep_i32[0].astype(bool)
    return k_mat, keep_mask


# speedup vs baseline: 726.3298x; 1.0640x over previous
import jax
import jax.numpy as jnp
from jax.experimental import pallas as pl
from jax.experimental.pallas import tpu as pltpu

M1 = 4096
M2 = 4096
D = 256

BM = 256
BN = 1024

LOG2E = 1.4426950408889634


def _tile_body(x1_ref, x2_ref, k_ref, keep_ref, bs_ref, a2_ref):
    j = pl.program_id(0)
    i = pl.program_id(1)

    @pl.when(i == 0)
    def _prep():
        b = x2_ref[...]
        bs_ref[...] = b * LOG2E
        a2_ref[...] = (-0.5 * LOG2E) * jnp.sum(b * b, axis=1)[None, :]
        keep_ref[...] = jnp.ones((1, BN), jnp.int32)

    a = x1_ref[...]
    a1 = (-0.5 * LOG2E) * jnp.sum(a * a, axis=1)
    p = jax.lax.dot_general(
        a, bs_ref[...], (((1,), (1,)), ((), ())),
        preferred_element_type=jnp.float32,
    )
    arg = p + a1[:, None] + a2_ref[...]
    k = jnp.minimum(jnp.exp2(arg), 1.0)
    k_ref[...] = k

    row_max = i * BM + BM - 1
    row_min = i * BM
    col_max = j * BN + BN - 1
    col_min = j * BN
    tile_all_upper = row_max <= col_min
    tile_all_lower = row_min > col_max

    @pl.when(tile_all_upper)
    def _full_reduce():
        m = jnp.logical_not(jnp.any(k >= 1.0, axis=0))[None, :]
        keep_ref[...] &= m.astype(jnp.int32)

    @pl.when(jnp.logical_not(tile_all_upper | tile_all_lower))
    def _diag_reduce():
        rows = row_min + jax.lax.broadcasted_iota(jnp.int32, (BM, BN), 0)
        cols = col_min + jax.lax.broadcasted_iota(jnp.int32, (BM, BN), 1)
        dup = (k >= 1.0) & (rows <= cols)
        m = jnp.logical_not(jnp.any(dup, axis=0))[None, :]
        keep_ref[...] &= m.astype(jnp.int32)


@jax.jit
def kernel(x1, x2):
    grid = (M2 // BN, M1 // BM)
    k_mat, keep_i32 = pl.pallas_call(
        _tile_body,
        grid=grid,
        in_specs=[
            pl.BlockSpec((BM, D), lambda j, i: (i, 0)),
            pl.BlockSpec((BN, D), lambda j, i: (j, 0)),
        ],
        out_specs=[
            pl.BlockSpec((BM, BN), lambda j, i: (i, j)),
            pl.BlockSpec((1, BN), lambda j, i: (0, j)),
        ],
        out_shape=[
            jax.ShapeDtypeStruct((M1, M2), jnp.float32),
            jax.ShapeDtypeStruct((1, M2), jnp.int32),
        ],
        scratch_shapes=[
            pltpu.VMEM((BN, D), jnp.float32),
            pltpu.VMEM((1, BN), jnp.float32),
        ],
        compiler_params=pltpu.CompilerParams(
            dimension_semantics=("parallel", "arbitrary"),
        ),
    )(x1, x2)
    keep_mask = keep_i32[0].astype(bool)
    return k_mat, keep_mask
